# baseline (speedup 1.0000x reference)
import jax
import jax.numpy as jnp
from jax.experimental import pallas as pl
from jax.experimental.pallas import tpu as pltpu

_DMODEL = 1024
_NE = 8
_ES = 128
_NT = 2048
_BLK = 512
_NBLK = _NT // _BLK


def _moe_body(x_ref, keys_ref, values_ref, es_ref, out_ref, reg_ref,
              kscr_ref, vmat_ref, h_ref, s_ref, sem_k, sem_v):
    i = pl.program_id(0)

    @pl.when(i == 0)
    def _():
        s_ref[...] = jnp.zeros_like(s_ref)
        for e in range(_NE):
            pltpu.make_async_copy(
                keys_ref.at[e], kscr_ref.at[e], sem_k.at[e]).start()
        for e in range(_NE):
            pltpu.make_async_copy(
                values_ref.at[e],
                vmat_ref.at[pl.ds(e * _ES, _ES), :], sem_v.at[e]).start()

    x = x_ref[...]
    sel_raw = jax.lax.dot_general(
        x, es_ref[...], (((1,), (1,)), ((), ())),
        preferred_element_type=jnp.float32)

    p = jnp.exp(sel_raw)
    p = p / jnp.sum(p, axis=-1, keepdims=True)
    s_ref[...] += jnp.sum(p, axis=0, keepdims=True)

    cols = jax.lax.broadcasted_iota(jnp.int32, sel_raw.shape, 1)
    idx1 = jnp.argmax(sel_raw, axis=-1)[:, None]
    v1 = jnp.max(sel_raw, axis=-1, keepdims=True)
    masked = jnp.where(cols == idx1, -jnp.inf, sel_raw)
    idx2 = jnp.argmax(masked, axis=-1)[:, None]
    v2 = jnp.max(masked, axis=-1, keepdims=True)
    g1 = jax.nn.sigmoid(v1)
    g2 = jax.nn.sigmoid(v2)

    for e in range(_NE):
        @pl.when(i == 0)
        def _(e=e):
            pltpu.make_async_copy(
                keys_ref.at[e], kscr_ref.at[e], sem_k.at[e]).wait()
        h_ref[:, e * _ES:(e + 1) * _ES] = jax.lax.dot_general(
            x, kscr_ref[e], (((1,), (0,)), ((), ())),
            preferred_element_type=jnp.float32)

    h = h_ref[...]
    ecol = jax.lax.broadcasted_iota(jnp.int32, h.shape, 1) >> 7
    w = (jnp.where(ecol == idx1, g1, 0.0)
         + jnp.where(ecol == idx2, g2, 0.0))
    h = jnp.maximum(h, 0.0) * w

    @pl.when(i == 0)
    def _():
        for e in range(_NE):
            pltpu.make_async_copy(
                values_ref.at[e],
                vmat_ref.at[pl.ds(e * _ES, _ES), :], sem_v.at[e]).wait()

    out_ref[...] = jax.lax.dot_general(
        h, vmat_ref[...], (((1,), (0,)), ((), ())),
        preferred_element_type=jnp.float32)

    @pl.when(i == _NBLK - 1)
    def _():
        s = s_ref[...]
        lm = jnp.log(s) - jnp.log(float(_NT))
        reg_ref[...] = jnp.sum(lm * (s / float(_NT)), axis=1, keepdims=True)


def kernel(x, keys, values, expert_sel):
    out, reg = pl.pallas_call(
        _moe_body,
        grid=(_NBLK,),
        in_specs=[
            pl.BlockSpec((_BLK, _DMODEL), lambda i: (i, 0)),
            pl.BlockSpec(memory_space=pl.ANY),
            pl.BlockSpec(memory_space=pl.ANY),
            pl.BlockSpec((_NE, _DMODEL), lambda i: (0, 0)),
        ],
        out_specs=[
            pl.BlockSpec((_BLK, _DMODEL), lambda i: (i, 0)),
            pl.BlockSpec((1, 1), lambda i: (0, 0)),
        ],
        out_shape=[
            jax.ShapeDtypeStruct((_NT, _DMODEL), jnp.float32),
            jax.ShapeDtypeStruct((1, 1), jnp.float32),
        ],
        scratch_shapes=[
            pltpu.VMEM((_NE, _DMODEL, _ES), jnp.float32),
            pltpu.VMEM((_NE * _ES, _DMODEL), jnp.float32),
            pltpu.VMEM((_BLK, _NE * _ES), jnp.float32),
            pltpu.VMEM((1, _NE), jnp.float32),
            pltpu.SemaphoreType.DMA((_NE,)),
            pltpu.SemaphoreType.DMA((_NE,)),
        ],
    )(x, keys, values, expert_sel)
    return out, reg[0, 0]
